# baseline (speedup 1.0000x reference)
import jax
import jax.numpy as jnp
from jax.experimental import pallas as pl
from jax.experimental.pallas import tpu as pltpu


def _fused_mlp_kernel(t0_ref, t1_ref, x_ref,
                      gva_ref, uva_ref, gla_ref, ula_ref, dva_ref, dla_ref,
                      gvb_ref, uvb_ref, glb_ref, ulb_ref, dvb_ref, dlb_ref,
                      out_ref, acc_ref):
    i = pl.program_id(0)
    steps = pl.num_programs(0) - 1
    f32 = jnp.float32

    def chunk_contrib(gv_ref, uv_ref, gl_ref, ul_ref, dv_ref, dl_ref):
        maskf = ((t0_ref[:] == 1) & (t1_ref[:] == 1)).astype(f32)
        x = x_ref[:]
        hv = jax.nn.silu(
            jnp.dot(x, gv_ref[:], preferred_element_type=f32)
        ) * jnp.dot(x, uv_ref[:], preferred_element_type=f32)
        hl = jax.nn.silu(
            jnp.dot(x, gl_ref[:], preferred_element_type=f32)
        ) * jnp.dot(x, ul_ref[:], preferred_element_type=f32)
        hv = hv * maskf
        hl = hl * (1.0 - maskf)
        contrib = (
            jnp.dot(hv, dv_ref[:], preferred_element_type=f32)
            + jnp.dot(hl, dl_ref[:], preferred_element_type=f32)
        )

        @pl.when(i == 1)
        def _():
            acc_ref[:] = contrib

        @pl.when(i > 1)
        def _():
            acc_ref[:] += contrib

    @pl.when((i >= 1) & (i % 2 == 1))
    def _():
        chunk_contrib(gva_ref, uva_ref, gla_ref, ula_ref, dva_ref, dla_ref)

    @pl.when((i >= 2) & (i % 2 == 0))
    def _():
        chunk_contrib(gvb_ref, uvb_ref, glb_ref, ulb_ref, dvb_ref, dlb_ref)

    @pl.when(i == steps)
    def _():
        out_ref[:] = acc_ref[:]


def kernel(hidden_states, token_type_ids, gate_v, up_v, down_v,
           gate_l, up_l, down_l):
    B, L, D = hidden_states.shape
    I = gate_v.shape[1]
    N = B * L
    C = 128
    chunks = I // C

    x = hidden_states.reshape(N, D)
    t0 = token_type_ids.reshape(N, 1)
    t_next = jnp.concatenate(
        [token_type_ids[:, 1:],
         jnp.full((B, 1), -1, dtype=token_type_ids.dtype)], axis=1)
    t1 = t_next.reshape(N, 1)

    def even_idx(i):
        return jnp.minimum(2 * (i // 2), chunks - 2)

    def odd_idx(i):
        return jnp.maximum(2 * ((i + 1) // 2) - 1, 1)

    gu_a = pl.BlockSpec((D, C), lambda i: (0, even_idx(i)))
    dn_a = pl.BlockSpec((C, D), lambda i: (even_idx(i), 0))
    gu_b = pl.BlockSpec((D, C), lambda i: (0, odd_idx(i)))
    dn_b = pl.BlockSpec((C, D), lambda i: (odd_idx(i), 0))

    out = pl.pallas_call(
        _fused_mlp_kernel,
        grid=(chunks + 1,),
        in_specs=[
            pl.BlockSpec((N, 1), lambda i: (0, 0)),
            pl.BlockSpec((N, 1), lambda i: (0, 0)),
            pl.BlockSpec((N, D), lambda i: (0, 0)),
            gu_a, gu_a, gu_a, gu_a, dn_a, dn_a,
            gu_b, gu_b, gu_b, gu_b, dn_b, dn_b,
        ],
        out_specs=pl.BlockSpec((N, D), lambda i: (0, 0)),
        out_shape=jax.ShapeDtypeStruct((N, D), jnp.float32),
        scratch_shapes=[
            pltpu.VMEM((N, D), jnp.float32),
        ],
        compiler_params=pltpu.CompilerParams(
            dimension_semantics=("arbitrary",),
        ),
    )(t0, t1, x,
      gate_v, up_v, gate_l, up_l, down_v, down_l,
      gate_v, up_v, gate_l, up_l, down_v, down_l)

    return out.reshape(B, L, D)


# speedup vs baseline: 1.0224x; 1.0224x over previous
import jax
import jax.numpy as jnp
from jax.experimental import pallas as pl
from jax.experimental.pallas import tpu as pltpu


def _fused_mlp_kernel(t0_ref, t1_ref, x_ref, gv_ref, uv_ref, gl_ref,
                      ul_ref, dv_ref, dl_ref, out_ref, acc_ref):
    i = pl.program_id(0)
    maskf = ((t0_ref[:] == 1) & (t1_ref[:] == 1)).astype(jnp.float32)

    f32 = jnp.float32
    x = x_ref[:]

    hv = jax.nn.silu(
        jnp.dot(x, gv_ref[:], preferred_element_type=f32)
    ) * jnp.dot(x, uv_ref[:], preferred_element_type=f32)
    hl = jax.nn.silu(
        jnp.dot(x, gl_ref[:], preferred_element_type=f32)
    ) * jnp.dot(x, ul_ref[:], preferred_element_type=f32)

    hv = hv * maskf
    hl = hl * (1.0 - maskf)
    contrib = (
        jnp.dot(hv, dv_ref[:], preferred_element_type=f32)
        + jnp.dot(hl, dl_ref[:], preferred_element_type=f32)
    )

    @pl.when(i == 0)
    def _():
        acc_ref[:] = contrib

    @pl.when(i > 0)
    def _():
        acc_ref[:] += contrib

    @pl.when(i == pl.num_programs(0) - 1)
    def _():
        out_ref[:] = acc_ref[:]


def kernel(hidden_states, token_type_ids, gate_v, up_v, down_v,
           gate_l, up_l, down_l):
    B, L, D = hidden_states.shape
    I = gate_v.shape[1]
    N = B * L
    C = 256
    steps = I // C

    x = hidden_states.reshape(N, D)
    t0 = token_type_ids.reshape(N, 1)
    t_next = jnp.concatenate(
        [token_type_ids[:, 1:],
         jnp.full((B, 1), -1, dtype=token_type_ids.dtype)], axis=1)
    t1 = t_next.reshape(N, 1)

    out = pl.pallas_call(
        _fused_mlp_kernel,
        grid=(steps,),
        in_specs=[
            pl.BlockSpec((N, 1), lambda i: (0, 0)),
            pl.BlockSpec((N, 1), lambda i: (0, 0)),
            pl.BlockSpec((N, D), lambda i: (0, 0)),
            pl.BlockSpec((D, C), lambda i: (0, i)),
            pl.BlockSpec((D, C), lambda i: (0, i)),
            pl.BlockSpec((D, C), lambda i: (0, i)),
            pl.BlockSpec((D, C), lambda i: (0, i)),
            pl.BlockSpec((C, D), lambda i: (i, 0)),
            pl.BlockSpec((C, D), lambda i: (i, 0)),
        ],
        out_specs=pl.BlockSpec((N, D), lambda i: (0, 0)),
        out_shape=jax.ShapeDtypeStruct((N, D), jnp.float32),
        scratch_shapes=[pltpu.VMEM((N, D), jnp.float32)],
        compiler_params=pltpu.CompilerParams(
            dimension_semantics=("arbitrary",),
        ),
    )(t0, t1, x, gate_v, up_v, gate_l, up_l, down_v, down_l)

    return out.reshape(B, L, D)
